# baseline (speedup 1.0000x reference)
import functools

import jax
import jax.numpy as jnp
import numpy as np
from jax import lax
from jax.experimental import pallas as pl
from jax.experimental.pallas import tpu as pltpu
from jax.experimental.pallas import tpu_sc as plsc

_B, _L, _S = 1024, 50, 32
_SUB = 32
_T = _L + 1
_BROWS = _T * _S
_CHUNKS = _T * 8
_V = 386
_VAUG = _V + _S
_NC, _NS = 2, 16
_NW = _NC * _NS
_BPW = _B // _NW


def _make_sc_kernel():
    mesh = plsc.VectorSubcoreMesh(core_axis_name="c", subcore_axis_name="s")

    @functools.partial(
        pl.kernel,
        mesh=mesh,
        out_type=jax.ShapeDtypeStruct((_B * _CHUNKS, 128), jnp.float32),
        scratch_types=[
            pltpu.VMEM((_BROWS,), jnp.int32),
            pltpu.VMEM((_BROWS,), jnp.int32),
            pltpu.VMEM((_BROWS, _SUB), jnp.float32),
            pltpu.VMEM((_BROWS, _SUB), jnp.float32),
            pltpu.VMEM_SHARED((_VAUG, _SUB), jnp.float32),
            pltpu.SemaphoreType.DMA,
            pltpu.SemaphoreType.DMA,
            pltpu.SemaphoreType.DMA,
            pltpu.SemaphoreType.DMA,
        ],
        compiler_params=pltpu.CompilerParams(use_tc_tiling_on_sc=False),
    )
    def k(xp_hbm, aug_hbm, out_hbm,
          idx_a, idx_b, rows_a, rows_b, aug_sh, gsem_a, gsem_b, isem_a, isem_b):
        wid = lax.axis_index("s") * _NC + lax.axis_index("c")
        b0 = wid * _BPW

        @pl.when(lax.axis_index("s") == 0)
        def _():
            pltpu.sync_copy(aug_hbm, aug_sh)

        plsc.subcore_barrier()

        def idx_copy(slot_ref, sem, i):
            return pltpu.make_async_copy(
                xp_hbm.at[pl.ds((b0 + i) * _BROWS, _BROWS)], slot_ref, sem)

        def gather(idx_ref, rows_ref, sem):
            return pltpu.make_async_copy(aug_sh.at[idx_ref], rows_ref, sem)

        def stores(rows_ref, sem, i):
            base = (b0 + i) * _CHUNKS
            return [
                pltpu.make_async_copy(
                    rows_ref.at[pl.ds(q * _CHUNKS, _CHUNKS)],
                    out_hbm.at[pl.ds(base, _CHUNKS), pl.ds(q * _SUB, _SUB)],
                    sem)
                for q in range(4)
            ]

        def store_all(rows_ref, sem, i):
            for c in stores(rows_ref, sem, i):
                c.start()
            for c in stores(rows_ref, sem, i):
                c.wait()

        idx_copy(idx_a, isem_a, 0).start()
        idx_copy(idx_b, isem_b, 1).start()
        idx_copy(idx_a, isem_a, 0).wait()
        gather(idx_a, rows_a, gsem_a).start()

        def body(j, carry):
            ia = 2 * j
            idx_copy(idx_b, isem_b, ia + 1).wait()
            gather(idx_b, rows_b, gsem_b).start()
            gather(idx_a, rows_a, gsem_a).wait()
            store_all(rows_a, gsem_a, ia)

            @pl.when(j < _BPW // 2 - 1)
            def _():
                idx_copy(idx_a, isem_a, ia + 2).start()
                idx_copy(idx_a, isem_a, ia + 2).wait()
                gather(idx_a, rows_a, gsem_a).start()
                idx_copy(idx_b, isem_b, ia + 3).start()

            gather(idx_b, rows_b, gsem_b).wait()
            store_all(rows_b, gsem_b, ia + 1)
            return carry

        lax.fori_loop(0, _BPW // 2, body, 0)

    return k


_sc_kernel = _make_sc_kernel()
_SOS_IDX = np.arange(_V, _VAUG, dtype=np.int32)


def _relayout_body(i_ref, o_ref):
    o_ref[...] = i_ref[...].reshape(o_ref.shape)


_relayout = pl.pallas_call(
    _relayout_body,
    grid=(128,),
    in_specs=[pl.BlockSpec((8 * _CHUNKS, 128), lambda b: (b, 0))],
    out_specs=pl.BlockSpec((8, _T, _S * _SUB), lambda b: (b, 0, 0)),
    out_shape=jax.ShapeDtypeStruct((_B, _T, _S * _SUB), jnp.float32),
)


def kernel(x, sos, table):
    xfull = jnp.concatenate(
        [jnp.broadcast_to(jnp.asarray(_SOS_IDX), (_B, _S)),
         x.reshape(_B, _L * _S).astype(jnp.int32)], axis=1)
    xp = xfull.reshape(_B, _T, 8, 4).transpose(0, 3, 1, 2).reshape(-1)
    aug = jnp.concatenate([table, sos.reshape(_S, _SUB)], axis=0)
    lin = _sc_kernel(xp, aug)
    return _relayout(lin)
